# baseline (speedup 1.0000x reference)
import functools

import jax
import jax.numpy as jnp
from jax import lax
from jax.experimental import pallas as pl
from jax.experimental.pallas import tpu as pltpu
from jax.experimental.pallas import tpu_sc as plsc

N = 10000
E = 320000
C = 128
HW = 80
DEG = 48
BN_EPS = 1e-5
LANES = 16
NC, NS = 2, 16
CH = 80
NCHUNK = E // CH
MAXCH = NCHUNK // NS
NB = 5
QO = 2
SO = 4
RQ = N // NS
BM = 512


def _tc_tables_body(x_ref, w1_ref, b1_ref, pp_ref, qq_ref):
    x = x_ref[...]
    w1 = w1_ref[...]
    wa = w1[:, :C]
    wb = w1[:, C:]
    dn = (((1,), (1,)), ((), ()))
    q = lax.dot_general(x, wb, dn, precision=lax.Precision.HIGHEST,
                        preferred_element_type=jnp.float32)
    p = lax.dot_general(x, wa - wb, dn, precision=lax.Precision.HIGHEST,
                        preferred_element_type=jnp.float32)
    p = p + b1_ref[...]
    bm = x.shape[0]
    iot = lax.broadcasted_iota(jnp.int32, (bm, 2 * HW - C), 1)
    deg_col = jnp.where(iot == 0, 1.0, 0.0).astype(jnp.float32)
    zpad = jnp.zeros((bm, 2 * HW - C), jnp.float32)
    pp_ref[0] = p[:, :HW]
    pp_ref[1] = jnp.concatenate([p[:, HW:], deg_col], axis=1)
    qq_ref[0] = q[:, :HW]
    qq_ref[1] = jnp.concatenate([q[:, HW:], zpad], axis=1)


def _tc_tables(x, w1, b1):
    grid = (pl.cdiv(N, BM),)
    pp, qq = pl.pallas_call(
        _tc_tables_body,
        grid=grid,
        in_specs=[
            pl.BlockSpec((BM, C), lambda i: (i, 0)),
            pl.BlockSpec((C, 2 * C), lambda i: (0, 0)),
            pl.BlockSpec((1, C), lambda i: (0, 0)),
        ],
        out_specs=[
            pl.BlockSpec((NC, BM, HW), lambda i: (0, i, 0)),
            pl.BlockSpec((NC, BM, HW), lambda i: (0, i, 0)),
        ],
        out_shape=[
            jax.ShapeDtypeStruct((NC, N, HW), jnp.float32),
            jax.ShapeDtypeStruct((NC, N, HW), jnp.float32),
        ],
    )(x, w1, b1.reshape(1, C))
    return pp, qq


def _tc_finalize_body(sp_ref, g_ref, beta_ref, w2_ref, b2_ref, o_ref, *, relu_norm):
    sb = sp_ref[...]
    s_lo = sb[0]
    s_hi = sb[1]
    s128 = jnp.concatenate([s_lo, s_hi[:, :C - HW]], axis=1)
    deg = s_hi[:, DEG:DEG + 1]
    sv = s128 * (g_ref[...] * (1.0 / jnp.sqrt(1.0 + BN_EPS)))
    dn = (((1,), (1,)), ((), ()))
    h = lax.dot_general(sv, w2_ref[...], dn, precision=lax.Precision.HIGHEST,
                        preferred_element_type=jnp.float32)
    cvec = lax.dot_general(beta_ref[...], w2_ref[...], dn,
                           precision=lax.Precision.HIGHEST,
                           preferred_element_type=jnp.float32)
    h = h + deg * (cvec + b2_ref[...])
    if relu_norm:
        h = jnp.maximum(h, 0.0)
        nrm = jnp.sqrt(jnp.sum(h * h, axis=1, keepdims=True))
        h = h / jnp.maximum(nrm, 1e-12)
    o_ref[...] = h


def _tc_finalize(sp, g, beta, w2, b2, relu_norm):
    grid = (pl.cdiv(N, BM),)
    return pl.pallas_call(
        functools.partial(_tc_finalize_body, relu_norm=relu_norm),
        grid=grid,
        in_specs=[
            pl.BlockSpec((NC, BM, HW), lambda i: (0, i, 0)),
            pl.BlockSpec((1, C), lambda i: (0, 0)),
            pl.BlockSpec((1, C), lambda i: (0, 0)),
            pl.BlockSpec((C, C), lambda i: (0, 0)),
            pl.BlockSpec((1, C), lambda i: (0, 0)),
        ],
        out_specs=pl.BlockSpec((BM, C), lambda i: (i, 0)),
        out_shape=jax.ShapeDtypeStruct((N, C), jnp.float32),
    )(sp, g.reshape(1, C), beta.reshape(1, C), w2, b2.reshape(1, C))


def _sc_edge_body(src_hbm, dst_hbm, pp_hbm, qq_hbm, out_hbm,
                  src_v, dst_v, bufs, ssum, sems):
    c = lax.axis_index("c")
    s = lax.axis_index("s")
    pp_c = pp_hbm.at[c]
    qq_c = qq_hbm.at[c]

    def zrow(r, _):
        for k in range(HW // LANES):
            bufs[0, r, pl.ds(k * LANES, LANES)] = jnp.zeros((LANES,), jnp.float32)
        return 0
    lax.fori_loop(0, CH, zrow, 0)
    row0 = s * RQ
    for t in range(RQ // CH):
        pltpu.sync_copy(bufs.at[0], ssum.at[pl.ds(row0 + t * CH, CH)])
    pltpu.sync_copy(bufs.at[0, pl.ds(0, RQ % CH)],
                    ssum.at[pl.ds(row0 + (RQ // CH) * CH, RQ % CH)])
    plsc.subcore_barrier()

    lo = s * MAXCH
    pltpu.sync_copy(src_hbm.at[pl.ds(lo, MAXCH)], src_v)
    pltpu.sync_copy(dst_hbm.at[pl.ds(lo, MAXCH)], dst_v)

    def wait_sem(b):
        pltpu.make_async_copy(pp_c.at[dst_v.at[0, 0]], bufs.at[b],
                              sems.at[b]).wait()

    def step(t, _):
        b_p = lax.rem(t, NB)
        b_q = lax.rem(t + NB - QO, NB)
        b_s = lax.rem(t + NB - SO, NB)

        @pl.when(jnp.logical_and(t >= NB, t < MAXCH))
        def _():
            wait_sem(b_p)

        @pl.when(t < MAXCH)
        def _():
            pltpu.async_copy(pp_c.at[dst_v.at[t, 0]], bufs.at[b_p],
                             sems.at[b_p])

        @pl.when(jnp.logical_and(t >= QO, t < MAXCH + QO))
        def _():
            wait_sem(b_q)
            pltpu.async_copy(qq_c.at[src_v.at[t - QO, 0]], bufs.at[b_q],
                             sems.at[b_q], add=True)

        @pl.when(t >= SO)
        def _():
            wait_sem(b_s)

            def _relu_rows(r, _):
                for j in range(4):
                    rr = r * 4 + j
                    for k in range(HW // LANES):
                        sl = pl.ds(k * LANES, LANES)
                        bufs[b_s, rr, sl] = jnp.maximum(bufs[b_s, rr, sl], 0.0)
                return 0
            lax.fori_loop(0, CH // 4, _relu_rows, 0)
            pltpu.async_copy(bufs.at[b_s], ssum.at[dst_v.at[t - SO, 0]],
                             sems.at[b_s], add=True)
        return 0
    lax.fori_loop(0, MAXCH + SO, step, 0)
    for b in range(NB):
        wait_sem(b)
    plsc.subcore_barrier()

    for t in range(RQ // CH):
        r0 = row0 + t * CH
        pltpu.sync_copy(ssum.at[pl.ds(r0, CH)], out_hbm.at[c, pl.ds(r0, CH)])
    r0 = row0 + (RQ // CH) * CH
    pltpu.sync_copy(ssum.at[pl.ds(r0, RQ % CH)],
                    out_hbm.at[c, pl.ds(r0, RQ % CH)])


_sc_edge_pass = pl.kernel(
    _sc_edge_body,
    out_type=jax.ShapeDtypeStruct((NC, N, HW), jnp.float32),
    mesh=plsc.VectorSubcoreMesh(core_axis_name="c", subcore_axis_name="s"),
    scratch_types=[
        pltpu.VMEM((MAXCH, 1, CH), jnp.int32),
        pltpu.VMEM((MAXCH, 1, CH), jnp.int32),
        pltpu.VMEM((NB, CH, HW), jnp.float32),
        pltpu.VMEM_SHARED((N, HW), jnp.float32),
        pltpu.SemaphoreType.DMA((NB,)),
    ],
    compiler_params=pltpu.CompilerParams(use_tc_tiling_on_sc=False),
)


def kernel(x, edge_index, edge_feature, W1_0, b1_0, g_0, beta_0, W2_0, b2_0,
           W1_1, b1_1, g_1, beta_1, W2_1, b2_1):
    src = edge_index[0].reshape(NCHUNK, 1, CH)
    dst = edge_index[1].reshape(NCHUNK, 1, CH)

    pp0, qq0 = _tc_tables(x, W1_0, b1_0)
    sp0 = _sc_edge_pass(src, dst, pp0, qq0)
    h = _tc_finalize(sp0, g_0, beta_0, W2_0, b2_0, True)

    pp1, qq1 = _tc_tables(h, W1_1, b1_1)
    sp1 = _sc_edge_pass(src, dst, pp1, qq1)
    return _tc_finalize(sp1, g_1, beta_1, W2_1, b2_1, False)


# speedup vs baseline: 1.1520x; 1.1520x over previous
import functools

import jax
import jax.numpy as jnp
from jax import lax
from jax.experimental import pallas as pl
from jax.experimental.pallas import tpu as pltpu
from jax.experimental.pallas import tpu_sc as plsc

N = 10000
E = 320000
C = 128
HW = 80
DEG = 48
BN_EPS = 1e-5
LANES = 16
NC, NS = 2, 16
CH = 160
NCHUNK = E // CH
MAXCH = NCHUNK // NS
NB = 3
QO = 1
SO = 2
RQ = N // NS
BM = 512


def _tc_tables_body(x_ref, w1_ref, b1_ref, pp_ref, qq_ref):
    x = x_ref[...]
    w1 = w1_ref[...]
    wa = w1[:, :C]
    wb = w1[:, C:]
    dn = (((1,), (1,)), ((), ()))
    q = lax.dot_general(x, wb, dn, precision=lax.Precision.HIGHEST,
                        preferred_element_type=jnp.float32)
    p = lax.dot_general(x, wa - wb, dn, precision=lax.Precision.HIGHEST,
                        preferred_element_type=jnp.float32)
    p = p + b1_ref[...]
    bm = x.shape[0]
    iot = lax.broadcasted_iota(jnp.int32, (bm, 2 * HW - C), 1)
    deg_col = jnp.where(iot == 0, 1.0, 0.0).astype(jnp.float32)
    zpad = jnp.zeros((bm, 2 * HW - C), jnp.float32)
    pp_ref[0] = p[:, :HW]
    pp_ref[1] = jnp.concatenate([p[:, HW:], deg_col], axis=1)
    qq_ref[0] = q[:, :HW]
    qq_ref[1] = jnp.concatenate([q[:, HW:], zpad], axis=1)


def _tc_tables(x, w1, b1):
    grid = (pl.cdiv(N, BM),)
    pp, qq = pl.pallas_call(
        _tc_tables_body,
        grid=grid,
        in_specs=[
            pl.BlockSpec((BM, C), lambda i: (i, 0)),
            pl.BlockSpec((C, 2 * C), lambda i: (0, 0)),
            pl.BlockSpec((1, C), lambda i: (0, 0)),
        ],
        out_specs=[
            pl.BlockSpec((NC, BM, HW), lambda i: (0, i, 0)),
            pl.BlockSpec((NC, BM, HW), lambda i: (0, i, 0)),
        ],
        out_shape=[
            jax.ShapeDtypeStruct((NC, N, HW), jnp.float32),
            jax.ShapeDtypeStruct((NC, N, HW), jnp.float32),
        ],
    )(x, w1, b1.reshape(1, C))
    return pp, qq


def _tc_finalize_body(sp_ref, g_ref, beta_ref, w2_ref, b2_ref, o_ref, *, relu_norm):
    sb = sp_ref[...]
    s_lo = sb[0]
    s_hi = sb[1]
    s128 = jnp.concatenate([s_lo, s_hi[:, :C - HW]], axis=1)
    deg = s_hi[:, DEG:DEG + 1]
    sv = s128 * (g_ref[...] * (1.0 / jnp.sqrt(1.0 + BN_EPS)))
    dn = (((1,), (1,)), ((), ()))
    h = lax.dot_general(sv, w2_ref[...], dn, precision=lax.Precision.HIGHEST,
                        preferred_element_type=jnp.float32)
    cvec = lax.dot_general(beta_ref[...], w2_ref[...], dn,
                           precision=lax.Precision.HIGHEST,
                           preferred_element_type=jnp.float32)
    h = h + deg * (cvec + b2_ref[...])
    if relu_norm:
        h = jnp.maximum(h, 0.0)
        nrm = jnp.sqrt(jnp.sum(h * h, axis=1, keepdims=True))
        h = h / jnp.maximum(nrm, 1e-12)
    o_ref[...] = h


def _tc_finalize(sp, g, beta, w2, b2, relu_norm):
    grid = (pl.cdiv(N, BM),)
    return pl.pallas_call(
        functools.partial(_tc_finalize_body, relu_norm=relu_norm),
        grid=grid,
        in_specs=[
            pl.BlockSpec((NC, BM, HW), lambda i: (0, i, 0)),
            pl.BlockSpec((1, C), lambda i: (0, 0)),
            pl.BlockSpec((1, C), lambda i: (0, 0)),
            pl.BlockSpec((C, C), lambda i: (0, 0)),
            pl.BlockSpec((1, C), lambda i: (0, 0)),
        ],
        out_specs=pl.BlockSpec((BM, C), lambda i: (i, 0)),
        out_shape=jax.ShapeDtypeStruct((N, C), jnp.float32),
    )(sp, g.reshape(1, C), beta.reshape(1, C), w2, b2.reshape(1, C))


def _sc_edge_body(src_hbm, dst_hbm, pp_hbm, qq_hbm, out_hbm,
                  src_v, dst_v, bufs, ssum, sems):
    c = lax.axis_index("c")
    s = lax.axis_index("s")
    pp_c = pp_hbm.at[c]
    qq_c = qq_hbm.at[c]

    def zrow(r, _):
        for k in range(HW // LANES):
            bufs[0, r, pl.ds(k * LANES, LANES)] = jnp.zeros((LANES,), jnp.float32)
        return 0
    lax.fori_loop(0, CH, zrow, 0)
    row0 = s * RQ
    for t in range(RQ // CH):
        pltpu.sync_copy(bufs.at[0], ssum.at[pl.ds(row0 + t * CH, CH)])
    pltpu.sync_copy(bufs.at[0, pl.ds(0, RQ % CH)],
                    ssum.at[pl.ds(row0 + (RQ // CH) * CH, RQ % CH)])
    plsc.subcore_barrier()

    lo = s * MAXCH
    pltpu.sync_copy(src_hbm.at[pl.ds(lo, MAXCH)], src_v)
    pltpu.sync_copy(dst_hbm.at[pl.ds(lo, MAXCH)], dst_v)

    def wait_sem(b):
        pltpu.make_async_copy(pp_c.at[dst_v.at[0, 0]], bufs.at[b],
                              sems.at[b]).wait()

    def step(t, _):
        b_p = lax.rem(t, NB)
        b_q = lax.rem(t + NB - QO, NB)
        b_s = lax.rem(t + NB - SO, NB)

        @pl.when(jnp.logical_and(t >= NB, t < MAXCH))
        def _():
            wait_sem(b_p)

        @pl.when(t < MAXCH)
        def _():
            pltpu.async_copy(pp_c.at[dst_v.at[t, 0]], bufs.at[b_p],
                             sems.at[b_p])

        @pl.when(jnp.logical_and(t >= QO, t < MAXCH + QO))
        def _():
            wait_sem(b_q)
            pltpu.async_copy(qq_c.at[src_v.at[t - QO, 0]], bufs.at[b_q],
                             sems.at[b_q], add=True)

        @pl.when(t >= SO)
        def _():
            wait_sem(b_s)

            def _relu_rows(r, _):
                for j in range(8):
                    rr = r * 8 + j
                    for k in range(HW // LANES):
                        sl = pl.ds(k * LANES, LANES)
                        bufs[b_s, rr, sl] = jnp.maximum(bufs[b_s, rr, sl], 0.0)
                return 0
            lax.fori_loop(0, CH // 8, _relu_rows, 0)
            pltpu.async_copy(bufs.at[b_s], ssum.at[dst_v.at[t - SO, 0]],
                             sems.at[b_s], add=True)
        return 0
    lax.fori_loop(0, MAXCH + SO, step, 0)
    for b in range(NB):
        wait_sem(b)
    plsc.subcore_barrier()

    for t in range(RQ // CH):
        r0 = row0 + t * CH
        pltpu.sync_copy(ssum.at[pl.ds(r0, CH)], out_hbm.at[c, pl.ds(r0, CH)])
    r0 = row0 + (RQ // CH) * CH
    pltpu.sync_copy(ssum.at[pl.ds(r0, RQ % CH)],
                    out_hbm.at[c, pl.ds(r0, RQ % CH)])


_sc_edge_pass = pl.kernel(
    _sc_edge_body,
    out_type=jax.ShapeDtypeStruct((NC, N, HW), jnp.float32),
    mesh=plsc.VectorSubcoreMesh(core_axis_name="c", subcore_axis_name="s"),
    scratch_types=[
        pltpu.VMEM((MAXCH, 1, CH), jnp.int32),
        pltpu.VMEM((MAXCH, 1, CH), jnp.int32),
        pltpu.VMEM((NB, CH, HW), jnp.float32),
        pltpu.VMEM_SHARED((N, HW), jnp.float32),
        pltpu.SemaphoreType.DMA((NB,)),
    ],
    compiler_params=pltpu.CompilerParams(use_tc_tiling_on_sc=False),
)


def kernel(x, edge_index, edge_feature, W1_0, b1_0, g_0, beta_0, W2_0, b2_0,
           W1_1, b1_1, g_1, beta_1, W2_1, b2_1):
    src = edge_index[0].reshape(NCHUNK, 1, CH)
    dst = edge_index[1].reshape(NCHUNK, 1, CH)

    pp0, qq0 = _tc_tables(x, W1_0, b1_0)
    sp0 = _sc_edge_pass(src, dst, pp0, qq0)
    h = _tc_finalize(sp0, g_0, beta_0, W2_0, b2_0, True)

    pp1, qq1 = _tc_tables(h, W1_1, b1_1)
    sp1 = _sc_edge_pass(src, dst, pp1, qq1)
    return _tc_finalize(sp1, g_1, beta_1, W2_1, b2_1, False)


# speedup vs baseline: 2.0082x; 1.7432x over previous
import functools

import jax
import jax.numpy as jnp
from jax import lax
from jax.experimental import pallas as pl
from jax.experimental.pallas import tpu as pltpu
from jax.experimental.pallas import tpu_sc as plsc

N = 10000
E = 320000
C = 128
HW = 80
DEG = 48
BN_EPS = 1e-5
LANES = 16
NC, NS = 2, 16
CH = 160
NCHUNK = E // CH
MAXCH = NCHUNK // NS
NB = 3
QO = 1
SO = 2
RQ = N // NS
BM = 512


def _tc_tables_body(x_ref, w1_ref, b1_ref, pp_ref, qq_ref):
    x = x_ref[...]
    w1 = w1_ref[...]
    wa = w1[:, :C]
    wb = w1[:, C:]
    dn = (((1,), (1,)), ((), ()))
    q = lax.dot_general(x, wb, dn, precision=lax.Precision.HIGHEST,
                        preferred_element_type=jnp.float32)
    p = lax.dot_general(x, wa - wb, dn, precision=lax.Precision.HIGHEST,
                        preferred_element_type=jnp.float32)
    p = p + b1_ref[...]
    bm = x.shape[0]
    iot = lax.broadcasted_iota(jnp.int32, (bm, 2 * HW - C), 1)
    deg_col = jnp.where(iot == 0, 1.0, 0.0).astype(jnp.float32)
    zpad = jnp.zeros((bm, 2 * HW - C), jnp.float32)
    pp_ref[0] = p[:, :HW]
    pp_ref[1] = jnp.concatenate([p[:, HW:], deg_col], axis=1)
    qq_ref[0] = q[:, :HW]
    qq_ref[1] = jnp.concatenate([q[:, HW:], zpad], axis=1)


def _tc_tables(x, w1, b1):
    grid = (pl.cdiv(N, BM),)
    pp, qq = pl.pallas_call(
        _tc_tables_body,
        grid=grid,
        in_specs=[
            pl.BlockSpec((BM, C), lambda i: (i, 0)),
            pl.BlockSpec((C, 2 * C), lambda i: (0, 0)),
            pl.BlockSpec((1, C), lambda i: (0, 0)),
        ],
        out_specs=[
            pl.BlockSpec((NC, BM, HW), lambda i: (0, i, 0)),
            pl.BlockSpec((NC, BM, HW), lambda i: (0, i, 0)),
        ],
        out_shape=[
            jax.ShapeDtypeStruct((NC, N, HW), jnp.float32),
            jax.ShapeDtypeStruct((NC, N, HW), jnp.float32),
        ],
    )(x, w1, b1.reshape(1, C))
    return pp, qq


def _tc_finalize_body(sp_ref, g_ref, beta_ref, w2_ref, b2_ref, o_ref, *, relu_norm):
    sb = sp_ref[...]
    s_lo = sb[0]
    s_hi = sb[1]
    s128 = jnp.concatenate([s_lo, s_hi[:, :C - HW]], axis=1)
    deg = s_hi[:, DEG:DEG + 1]
    sv = s128 * (g_ref[...] * (1.0 / jnp.sqrt(1.0 + BN_EPS)))
    dn = (((1,), (1,)), ((), ()))
    h = lax.dot_general(sv, w2_ref[...], dn, precision=lax.Precision.HIGHEST,
                        preferred_element_type=jnp.float32)
    cvec = lax.dot_general(beta_ref[...], w2_ref[...], dn,
                           precision=lax.Precision.HIGHEST,
                           preferred_element_type=jnp.float32)
    h = h + deg * (cvec + b2_ref[...])
    if relu_norm:
        h = jnp.maximum(h, 0.0)
        nrm = jnp.sqrt(jnp.sum(h * h, axis=1, keepdims=True))
        h = h / jnp.maximum(nrm, 1e-12)
    o_ref[...] = h


def _tc_finalize(sp, g, beta, w2, b2, relu_norm):
    grid = (pl.cdiv(N, BM),)
    return pl.pallas_call(
        functools.partial(_tc_finalize_body, relu_norm=relu_norm),
        grid=grid,
        in_specs=[
            pl.BlockSpec((NC, BM, HW), lambda i: (0, i, 0)),
            pl.BlockSpec((1, C), lambda i: (0, 0)),
            pl.BlockSpec((1, C), lambda i: (0, 0)),
            pl.BlockSpec((C, C), lambda i: (0, 0)),
            pl.BlockSpec((1, C), lambda i: (0, 0)),
        ],
        out_specs=pl.BlockSpec((BM, C), lambda i: (i, 0)),
        out_shape=jax.ShapeDtypeStruct((N, C), jnp.float32),
    )(sp, g.reshape(1, C), beta.reshape(1, C), w2, b2.reshape(1, C))


def _sc_edge_body(src_hbm, dst_hbm, pp_hbm, qq_hbm, out_hbm,
                  src_v, dst_v, bufs, ssum, sems):
    c = lax.axis_index("c")
    s = lax.axis_index("s")
    pp_c = pp_hbm.at[c]
    qq_c = qq_hbm.at[c]

    def zrow(r, _):
        for k in range(HW // LANES):
            bufs[0, r, pl.ds(k * LANES, LANES)] = jnp.zeros((LANES,), jnp.float32)
        return 0
    lax.fori_loop(0, CH, zrow, 0)
    row0 = s * RQ
    for t in range(RQ // CH):
        pltpu.sync_copy(bufs.at[0], ssum.at[pl.ds(row0 + t * CH, CH)])
    pltpu.sync_copy(bufs.at[0, pl.ds(0, RQ % CH)],
                    ssum.at[pl.ds(row0 + (RQ // CH) * CH, RQ % CH)])
    plsc.subcore_barrier()

    lo = s * MAXCH
    pltpu.sync_copy(src_hbm.at[pl.ds(lo, MAXCH)], src_v)
    pltpu.sync_copy(dst_hbm.at[pl.ds(lo, MAXCH)], dst_v)

    def wait_sem(b):
        pltpu.make_async_copy(pp_c.at[dst_v.at[0, 0]], bufs.at[b],
                              sems.at[b]).wait()

    def step(t, _):
        b_p = lax.rem(t, NB)
        b_q = lax.rem(t + NB - QO, NB)
        b_s = lax.rem(t + NB - SO, NB)

        @pl.when(jnp.logical_and(t >= NB, t < MAXCH))
        def _():
            wait_sem(b_p)

        @pl.when(t < MAXCH)
        def _():
            pltpu.async_copy(pp_c.at[dst_v.at[t, 0]], bufs.at[b_p],
                             sems.at[b_p])

        @pl.when(jnp.logical_and(t >= QO, t < MAXCH + QO))
        def _():
            wait_sem(b_q)
            pltpu.async_copy(qq_c.at[src_v.at[t - QO, 0]], bufs.at[b_q],
                             sems.at[b_q], add=True)

        @pl.when(t >= SO)
        def _():
            wait_sem(b_s)

            def _relu_rows(r, _):
                vals = []
                for j in range(4):
                    rr = r * 4 + j
                    for k in range(HW // LANES):
                        sl = pl.ds(k * LANES, LANES)
                        vals.append((rr, sl, jnp.maximum(bufs[b_s, rr, sl], 0.0)))
                for rr, sl, v in vals:
                    bufs[b_s, rr, sl] = v
                return 0
            lax.fori_loop(0, CH // 4, _relu_rows, 0)
            pltpu.async_copy(bufs.at[b_s], ssum.at[dst_v.at[t - SO, 0]],
                             sems.at[b_s], add=True)
        return 0
    lax.fori_loop(0, MAXCH + SO, step, 0)
    for b in range(NB):
        wait_sem(b)
    plsc.subcore_barrier()

    for t in range(RQ // CH):
        r0 = row0 + t * CH
        pltpu.sync_copy(ssum.at[pl.ds(r0, CH)], out_hbm.at[c, pl.ds(r0, CH)])
    r0 = row0 + (RQ // CH) * CH
    pltpu.sync_copy(ssum.at[pl.ds(r0, RQ % CH)],
                    out_hbm.at[c, pl.ds(r0, RQ % CH)])


_sc_edge_pass = pl.kernel(
    _sc_edge_body,
    out_type=jax.ShapeDtypeStruct((NC, N, HW), jnp.float32),
    mesh=plsc.VectorSubcoreMesh(core_axis_name="c", subcore_axis_name="s"),
    scratch_types=[
        pltpu.VMEM((MAXCH, 1, CH), jnp.int32),
        pltpu.VMEM((MAXCH, 1, CH), jnp.int32),
        pltpu.VMEM((NB, CH, HW), jnp.float32),
        pltpu.VMEM_SHARED((N, HW), jnp.float32),
        pltpu.SemaphoreType.DMA((NB,)),
    ],
    compiler_params=pltpu.CompilerParams(use_tc_tiling_on_sc=False),
)


def kernel(x, edge_index, edge_feature, W1_0, b1_0, g_0, beta_0, W2_0, b2_0,
           W1_1, b1_1, g_1, beta_1, W2_1, b2_1):
    src = edge_index[0].reshape(NCHUNK, 1, CH)
    dst = edge_index[1].reshape(NCHUNK, 1, CH)

    pp0, qq0 = _tc_tables(x, W1_0, b1_0)
    sp0 = _sc_edge_pass(src, dst, pp0, qq0)
    h = _tc_finalize(sp0, g_0, beta_0, W2_0, b2_0, True)

    pp1, qq1 = _tc_tables(h, W1_1, b1_1)
    sp1 = _sc_edge_pass(src, dst, pp1, qq1)
    return _tc_finalize(sp1, g_1, beta_1, W2_1, b2_1, False)


# speedup vs baseline: 2.0419x; 1.0167x over previous
import functools

import jax
import jax.numpy as jnp
from jax import lax
from jax.experimental import pallas as pl
from jax.experimental.pallas import tpu as pltpu
from jax.experimental.pallas import tpu_sc as plsc

N = 10000
E = 320000
C = 128
HW = 80
DEG = 48
BN_EPS = 1e-5
LANES = 16
NC, NS = 2, 16
CH = 160
NCHUNK = E // CH
MAXCH = NCHUNK // NS
NB = 3
QO = 1
SO = 2
RQ = N // NS
BM = 512


def _tc_tables_common(x, w1_ref, b1_ref, pp_ref, qq_ref):
    w1 = w1_ref[...]
    wa = w1[:, :C]
    wb = w1[:, C:]
    dn = (((1,), (1,)), ((), ()))
    q = lax.dot_general(x, wb, dn, precision=lax.Precision.HIGHEST,
                        preferred_element_type=jnp.float32)
    p = lax.dot_general(x, wa - wb, dn, precision=lax.Precision.HIGHEST,
                        preferred_element_type=jnp.float32)
    p = p + b1_ref[...]
    bm = x.shape[0]
    iot = lax.broadcasted_iota(jnp.int32, (bm, 2 * HW - C), 1)
    deg_col = jnp.where(iot == 0, 1.0, 0.0).astype(jnp.float32)
    zpad = jnp.zeros((bm, 2 * HW - C), jnp.float32)
    pp_ref[0] = p[:, :HW]
    pp_ref[1] = jnp.concatenate([p[:, HW:], deg_col], axis=1)
    qq_ref[0] = q[:, :HW]
    qq_ref[1] = jnp.concatenate([q[:, HW:], zpad], axis=1)


def _tc_tables_body(x_ref, w1_ref, b1_ref, pp_ref, qq_ref):
    _tc_tables_common(x_ref[...], w1_ref, b1_ref, pp_ref, qq_ref)


def _tc_tables(x, w1, b1):
    grid = (pl.cdiv(N, BM),)
    pp, qq = pl.pallas_call(
        _tc_tables_body,
        grid=grid,
        in_specs=[
            pl.BlockSpec((BM, C), lambda i: (i, 0)),
            pl.BlockSpec((C, 2 * C), lambda i: (0, 0)),
            pl.BlockSpec((1, C), lambda i: (0, 0)),
        ],
        out_specs=[
            pl.BlockSpec((NC, BM, HW), lambda i: (0, i, 0)),
            pl.BlockSpec((NC, BM, HW), lambda i: (0, i, 0)),
        ],
        out_shape=[
            jax.ShapeDtypeStruct((NC, N, HW), jnp.float32),
            jax.ShapeDtypeStruct((NC, N, HW), jnp.float32),
        ],
    )(x, w1, b1.reshape(1, C))
    return pp, qq


def _tc_finalize_body(sp_ref, g_ref, beta_ref, w2_ref, b2_ref, o_ref, *, relu_norm):
    sb = sp_ref[...]
    s_lo = sb[0]
    s_hi = sb[1]
    s128 = jnp.concatenate([s_lo, s_hi[:, :C - HW]], axis=1)
    deg = s_hi[:, DEG:DEG + 1]
    sv = s128 * (g_ref[...] * (1.0 / jnp.sqrt(1.0 + BN_EPS)))
    dn = (((1,), (1,)), ((), ()))
    h = lax.dot_general(sv, w2_ref[...], dn, precision=lax.Precision.HIGHEST,
                        preferred_element_type=jnp.float32)
    cvec = lax.dot_general(beta_ref[...], w2_ref[...], dn,
                           precision=lax.Precision.HIGHEST,
                           preferred_element_type=jnp.float32)
    h = h + deg * (cvec + b2_ref[...])
    if relu_norm:
        h = jnp.maximum(h, 0.0)
        nrm = jnp.sqrt(jnp.sum(h * h, axis=1, keepdims=True))
        h = h / jnp.maximum(nrm, 1e-12)
    o_ref[...] = h


def _tc_finalize(sp, g, beta, w2, b2, relu_norm):
    grid = (pl.cdiv(N, BM),)
    return pl.pallas_call(
        functools.partial(_tc_finalize_body, relu_norm=relu_norm),
        grid=grid,
        in_specs=[
            pl.BlockSpec((NC, BM, HW), lambda i: (0, i, 0)),
            pl.BlockSpec((1, C), lambda i: (0, 0)),
            pl.BlockSpec((1, C), lambda i: (0, 0)),
            pl.BlockSpec((C, C), lambda i: (0, 0)),
            pl.BlockSpec((1, C), lambda i: (0, 0)),
        ],
        out_specs=pl.BlockSpec((BM, C), lambda i: (i, 0)),
        out_shape=jax.ShapeDtypeStruct((N, C), jnp.float32),
    )(sp, g.reshape(1, C), beta.reshape(1, C), w2, b2.reshape(1, C))


def _tc_fin_tab_body(sp_ref, g_ref, beta_ref, w2_ref, b2_ref,
                     w1_ref, b1_ref, pp_ref, qq_ref):
    sb = sp_ref[...]
    s128 = jnp.concatenate([sb[0], sb[1][:, :C - HW]], axis=1)
    deg = sb[1][:, DEG:DEG + 1]
    sv = s128 * (g_ref[...] * (1.0 / jnp.sqrt(1.0 + BN_EPS)))
    dn = (((1,), (1,)), ((), ()))
    h = lax.dot_general(sv, w2_ref[...], dn, precision=lax.Precision.HIGHEST,
                        preferred_element_type=jnp.float32)
    cvec = lax.dot_general(beta_ref[...], w2_ref[...], dn,
                           precision=lax.Precision.HIGHEST,
                           preferred_element_type=jnp.float32)
    h = h + deg * (cvec + b2_ref[...])
    h = jnp.maximum(h, 0.0)
    nrm = jnp.sqrt(jnp.sum(h * h, axis=1, keepdims=True))
    h = h / jnp.maximum(nrm, 1e-12)
    _tc_tables_common(h, w1_ref, b1_ref, pp_ref, qq_ref)


def _tc_fin_tab(sp, g, beta, w2, b2, w1, b1):
    grid = (pl.cdiv(N, BM),)
    return pl.pallas_call(
        _tc_fin_tab_body,
        grid=grid,
        in_specs=[
            pl.BlockSpec((NC, BM, HW), lambda i: (0, i, 0)),
            pl.BlockSpec((1, C), lambda i: (0, 0)),
            pl.BlockSpec((1, C), lambda i: (0, 0)),
            pl.BlockSpec((C, C), lambda i: (0, 0)),
            pl.BlockSpec((1, C), lambda i: (0, 0)),
            pl.BlockSpec((C, 2 * C), lambda i: (0, 0)),
            pl.BlockSpec((1, C), lambda i: (0, 0)),
        ],
        out_specs=[
            pl.BlockSpec((NC, BM, HW), lambda i: (0, i, 0)),
            pl.BlockSpec((NC, BM, HW), lambda i: (0, i, 0)),
        ],
        out_shape=[
            jax.ShapeDtypeStruct((NC, N, HW), jnp.float32),
            jax.ShapeDtypeStruct((NC, N, HW), jnp.float32),
        ],
    )(sp, g.reshape(1, C), beta.reshape(1, C), w2, b2.reshape(1, C),
      w1, b1.reshape(1, C))


def _sc_edge_body(src_hbm, dst_hbm, pp_hbm, qq_hbm, out_hbm,
                  src_v, dst_v, bufs, ssum, sems):
    c = lax.axis_index("c")
    s = lax.axis_index("s")
    pp_c = pp_hbm.at[c]
    qq_c = qq_hbm.at[c]

    def zrow(r, _):
        for k in range(HW // LANES):
            bufs[0, r, pl.ds(k * LANES, LANES)] = jnp.zeros((LANES,), jnp.float32)
        return 0
    lax.fori_loop(0, CH, zrow, 0)
    row0 = s * RQ
    for t in range(RQ // CH):
        pltpu.sync_copy(bufs.at[0], ssum.at[pl.ds(row0 + t * CH, CH)])
    pltpu.sync_copy(bufs.at[0, pl.ds(0, RQ % CH)],
                    ssum.at[pl.ds(row0 + (RQ // CH) * CH, RQ % CH)])
    plsc.subcore_barrier()

    lo = s * MAXCH
    pltpu.sync_copy(src_hbm.at[pl.ds(lo, MAXCH)], src_v)
    pltpu.sync_copy(dst_hbm.at[pl.ds(lo, MAXCH)], dst_v)

    def wait_sem(b):
        pltpu.make_async_copy(pp_c.at[dst_v.at[0, 0]], bufs.at[b],
                              sems.at[b]).wait()

    def step(t, _):
        b_p = lax.rem(t, NB)
        b_q = lax.rem(t + NB - QO, NB)
        b_s = lax.rem(t + NB - SO, NB)

        @pl.when(jnp.logical_and(t >= NB, t < MAXCH))
        def _():
            wait_sem(b_p)

        @pl.when(t < MAXCH)
        def _():
            pltpu.async_copy(pp_c.at[dst_v.at[t, 0]], bufs.at[b_p],
                             sems.at[b_p])

        @pl.when(jnp.logical_and(t >= QO, t < MAXCH + QO))
        def _():
            wait_sem(b_q)
            pltpu.async_copy(qq_c.at[src_v.at[t - QO, 0]], bufs.at[b_q],
                             sems.at[b_q], add=True)

        @pl.when(t >= SO)
        def _():
            wait_sem(b_s)

            def _relu_rows(r, _):
                vals = []
                for j in range(4):
                    rr = r * 4 + j
                    for k in range(HW // LANES):
                        sl = pl.ds(k * LANES, LANES)
                        vals.append((rr, sl, jnp.maximum(bufs[b_s, rr, sl], 0.0)))
                for rr, sl, v in vals:
                    bufs[b_s, rr, sl] = v
                return 0
            lax.fori_loop(0, CH // 4, _relu_rows, 0)
            pltpu.async_copy(bufs.at[b_s], ssum.at[dst_v.at[t - SO, 0]],
                             sems.at[b_s], add=True)
        return 0
    lax.fori_loop(0, MAXCH + SO, step, 0)
    for b in range(NB):
        wait_sem(b)
    plsc.subcore_barrier()

    for t in range(RQ // CH):
        r0 = row0 + t * CH
        pltpu.sync_copy(ssum.at[pl.ds(r0, CH)], out_hbm.at[c, pl.ds(r0, CH)])
    r0 = row0 + (RQ // CH) * CH
    pltpu.sync_copy(ssum.at[pl.ds(r0, RQ % CH)],
                    out_hbm.at[c, pl.ds(r0, RQ % CH)])


_sc_edge_pass = pl.kernel(
    _sc_edge_body,
    out_type=jax.ShapeDtypeStruct((NC, N, HW), jnp.float32),
    mesh=plsc.VectorSubcoreMesh(core_axis_name="c", subcore_axis_name="s"),
    scratch_types=[
        pltpu.VMEM((MAXCH, 1, CH), jnp.int32),
        pltpu.VMEM((MAXCH, 1, CH), jnp.int32),
        pltpu.VMEM((NB, CH, HW), jnp.float32),
        pltpu.VMEM_SHARED((N, HW), jnp.float32),
        pltpu.SemaphoreType.DMA((NB,)),
    ],
    compiler_params=pltpu.CompilerParams(use_tc_tiling_on_sc=False),
)


def kernel(x, edge_index, edge_feature, W1_0, b1_0, g_0, beta_0, W2_0, b2_0,
           W1_1, b1_1, g_1, beta_1, W2_1, b2_1):
    src = edge_index[0].reshape(NCHUNK, 1, CH)
    dst = edge_index[1].reshape(NCHUNK, 1, CH)

    pp0, qq0 = _tc_tables(x, W1_0, b1_0)
    sp0 = _sc_edge_pass(src, dst, pp0, qq0)
    pp1, qq1 = _tc_fin_tab(sp0, g_0, beta_0, W2_0, b2_0, W1_1, b1_1)
    sp1 = _sc_edge_pass(src, dst, pp1, qq1)
    return _tc_finalize(sp1, g_1, beta_1, W2_1, b2_1, False)
